# baseline (speedup 1.0000x reference)
import jax
import jax.numpy as jnp
from jax.experimental import pallas as pl
from jax.experimental.pallas import tpu as pltpu

P = 4096
B = 6
F_IN = 960
NCOL = 12
NS = 6 * B


def _proj_kernel(emb_ref, refc_ref, w_emb_ref, w_ref_ref, t_ref):
    t_ref[...] = (
        jnp.dot(emb_ref[...], w_emb_ref[...], preferred_element_type=jnp.float32)
        + jnp.dot(refc_ref[...], w_ref_ref[...], preferred_element_type=jnp.float32)
    )


def _agg_kernel(adj_ref, sup_ref, self_ref, bias_ref, out_ref, *, nq):
    q = pl.program_id(1)
    part = jnp.dot(adj_ref[...], sup_ref[...], preferred_element_type=jnp.float32)

    @pl.when(q == 0)
    def _init():
        out_ref[...] = part

    @pl.when(q > 0)
    def _accum():
        out_ref[...] = out_ref[...] + part

    @pl.when(q == nq - 1)
    def _finish():
        x = out_ref[...] + self_ref[...] + bias_ref[...]
        col = jax.lax.broadcasted_iota(jnp.int32, x.shape, 1)
        out_ref[...] = jnp.where((col % 6) < 3, jnp.tanh(x), jax.nn.sigmoid(x))


def kernel(embeddings, ref, adj, W_d, Wl_d, b_d, W_r, Wl_r, b_r):
    emb2 = embeddings.reshape(B * P, F_IN)
    refc = ref.reshape(P, 3)
    W_all = jnp.concatenate([W_d, W_r, Wl_d, Wl_r], axis=1)
    W_emb = W_all[:F_IN]
    W_ref = W_all[F_IN:]

    PB = 1024
    nb = (B * P) // PB
    rb = P // PB
    T = pl.pallas_call(
        _proj_kernel,
        grid=(nb,),
        in_specs=[
            pl.BlockSpec((PB, F_IN), lambda i: (i, 0)),
            pl.BlockSpec((PB, 3), lambda i: (jax.lax.rem(i, rb), 0)),
            pl.BlockSpec((F_IN, NCOL), lambda i: (0, 0)),
            pl.BlockSpec((3, NCOL), lambda i: (0, 0)),
        ],
        out_specs=pl.BlockSpec((PB, NCOL), lambda i: (i, 0)),
        out_shape=jax.ShapeDtypeStruct((B * P, NCOL), jnp.float32),
    )(emb2, refc, W_emb, W_ref)

    T3 = T.reshape(B, P, NCOL)
    sup = T3[:, :, 0:6].transpose(1, 0, 2).reshape(P, NS)
    slf = T3[:, :, 6:12].transpose(1, 0, 2).reshape(P, NS)
    bias = jnp.tile(jnp.concatenate([b_d, b_r]), B).reshape(1, NS)

    PBLK = 256
    QBLK = 1024
    npb, nq = P // PBLK, P // QBLK
    out36 = pl.pallas_call(
        lambda a, s, f, bz, o: _agg_kernel(a, s, f, bz, o, nq=nq),
        grid=(npb, nq),
        in_specs=[
            pl.BlockSpec((PBLK, QBLK), lambda p, q: (p, q)),
            pl.BlockSpec((QBLK, NS), lambda p, q: (q, 0)),
            pl.BlockSpec((PBLK, NS), lambda p, q: (p, 0)),
            pl.BlockSpec((1, NS), lambda p, q: (0, 0)),
        ],
        out_specs=pl.BlockSpec((PBLK, NS), lambda p, q: (p, 0)),
        out_shape=jax.ShapeDtypeStruct((P, NS), jnp.float32),
        compiler_params=pltpu.CompilerParams(
            dimension_semantics=("arbitrary", "arbitrary"),
        ),
    )(adj, sup, slf, bias)

    out3 = out36.reshape(P, B, 6).transpose(1, 0, 2)
    return out3[:, :, 0:3], out3[:, :, 3:6]
